# baseline (speedup 1.0000x reference)
import jax
import jax.numpy as jnp
from jax.experimental import pallas as pl

_DIM = 512
_E = 16
_HID = 32


def _moe_body(x_ref, gw_ref, gb_ref, w1_ref, b1_ref, w2_ref, b2_ref,
              sw1_ref, sb1_ref, sw2_ref, sb2_ref, o_ref):
    x = x_ref[...]
    f32 = jnp.float32

    logits = jnp.dot(x, gw_ref[...], preferred_element_type=f32) + gb_ref[...]
    m = jnp.max(logits, axis=-1, keepdims=True)
    p = jnp.exp(logits - m)
    probs = p / jnp.sum(p, axis=-1, keepdims=True)

    e_iota = jax.lax.broadcasted_iota(jnp.int32, probs.shape, 1)
    big = jnp.int32(_E)

    v1 = jnp.max(probs, axis=-1, keepdims=True)
    idx1 = jnp.min(jnp.where(probs == v1, e_iota, big), axis=-1, keepdims=True)
    one1 = (e_iota == idx1).astype(f32)
    probs2 = probs - one1 * 2.0
    v2 = jnp.max(probs2, axis=-1, keepdims=True)
    idx2 = jnp.min(jnp.where(probs2 == v2, e_iota, big), axis=-1, keepdims=True)
    one2 = (e_iota == idx2).astype(f32)

    c1 = jnp.sum(one1, axis=0, keepdims=True)
    c2 = jnp.sum(one2, axis=0, keepdims=True)

    col_e = jax.lax.broadcasted_iota(jnp.int32, (_E, _DIM), 1) // _HID
    row_e = jax.lax.broadcasted_iota(jnp.int32, (_E, _DIM), 0)
    rep = (col_e == row_e).astype(f32)
    c1rep = jnp.dot(c1, rep, preferred_element_type=f32)
    c2rep = jnp.dot(c2, rep, preferred_element_type=f32)
    scale = v1 * c1rep + v2 * c2rep

    h_parts = [jnp.dot(x, w1_ref[e], preferred_element_type=f32)
               for e in range(_E)]
    h = jnp.maximum(jnp.concatenate(h_parts, axis=1) + b1_ref[...], 0.0)
    routed = jnp.dot(h * scale, w2_ref[...], preferred_element_type=f32)
    bias_row = (v1 * jnp.dot(c1, b2_ref[...], preferred_element_type=f32)
                + v2 * jnp.dot(c2, b2_ref[...], preferred_element_type=f32))

    sh0 = jnp.maximum(jnp.dot(x, sw1_ref[0], preferred_element_type=f32)
                      + sb1_ref[0:1, :], 0.0)
    acc = jnp.dot(sh0, sw2_ref[0], preferred_element_type=f32) + sb2_ref[0:1, :]
    sh1 = jnp.maximum(jnp.dot(x, sw1_ref[1], preferred_element_type=f32)
                      + sb1_ref[1:2, :], 0.0)
    acc = acc + jnp.dot(sh1, sw2_ref[1], preferred_element_type=f32) + sb2_ref[1:2, :]

    o_ref[...] = acc + routed + bias_row


def kernel(x, gate_w, gate_b, W1, B1, W2, B2, SW1, SB1, SW2, SB2):
    b, s, d = x.shape
    x_flat = x.reshape(-1, d)
    b1cat = B1.reshape(1, _E * _HID)
    w2cat = W2.reshape(_E * _HID, d)
    gb = gate_b.reshape(1, -1)

    out = pl.pallas_call(
        _moe_body,
        out_shape=jax.ShapeDtypeStruct((x_flat.shape[0], d), jnp.float32),
    )(x_flat, gate_w, gb, W1, b1cat, w2cat, B2,
      SW1, SB1, SW2, SB2)
    return out.reshape(b, s, d)


# speedup vs baseline: 1.3357x; 1.3357x over previous
import jax
import jax.numpy as jnp
from jax.experimental import pallas as pl
from jax.experimental.pallas import tpu as pltpu

_DIM = 512
_E = 16
_HID = 32


def _moe_body(x_ref, gw_ref, gb_ref, w1_hbm, b1_ref, w2_hbm, b2_ref,
              sw1_hbm, sb1_ref, sw2_hbm, sb2_ref, o_ref,
              w1_s, w2_s, sw1_s, sw2_s, sems):
    f32 = jnp.float32

    cp_w1 = pltpu.make_async_copy(w1_hbm, w1_s, sems.at[0])
    cp_w1.start()
    cp_w2 = pltpu.make_async_copy(w2_hbm, w2_s, sems.at[1])
    cp_w2.start()
    cp_sw1a = pltpu.make_async_copy(sw1_hbm.at[0], sw1_s.at[0], sems.at[2])
    cp_sw1a.start()
    cp_sw2a = pltpu.make_async_copy(sw2_hbm.at[0], sw2_s.at[0], sems.at[3])
    cp_sw2a.start()
    cp_sw1b = pltpu.make_async_copy(sw1_hbm.at[1], sw1_s.at[1], sems.at[4])
    cp_sw1b.start()
    cp_sw2b = pltpu.make_async_copy(sw2_hbm.at[1], sw2_s.at[1], sems.at[5])
    cp_sw2b.start()

    x = x_ref[...]

    logits = jnp.dot(x, gw_ref[...], preferred_element_type=f32) + gb_ref[...]
    m = jnp.max(logits, axis=-1, keepdims=True)
    p = jnp.exp(logits - m)
    probs = p / jnp.sum(p, axis=-1, keepdims=True)

    e_iota = jax.lax.broadcasted_iota(jnp.int32, probs.shape, 1)
    big = jnp.int32(_E)

    v1 = jnp.max(probs, axis=-1, keepdims=True)
    idx1 = jnp.min(jnp.where(probs == v1, e_iota, big), axis=-1, keepdims=True)
    one1 = (e_iota == idx1).astype(f32)
    probs2 = probs - one1 * 2.0
    v2 = jnp.max(probs2, axis=-1, keepdims=True)
    idx2 = jnp.min(jnp.where(probs2 == v2, e_iota, big), axis=-1, keepdims=True)
    one2 = (e_iota == idx2).astype(f32)

    c1 = jnp.sum(one1, axis=0, keepdims=True)
    c2 = jnp.sum(one2, axis=0, keepdims=True)

    col_e = jax.lax.broadcasted_iota(jnp.int32, (_E, _DIM), 1) // _HID
    row_e = jax.lax.broadcasted_iota(jnp.int32, (_E, _DIM), 0)
    rep = (col_e == row_e).astype(f32)
    c1rep = jnp.dot(c1, rep, preferred_element_type=f32)
    c2rep = jnp.dot(c2, rep, preferred_element_type=f32)
    scale = v1 * c1rep + v2 * c2rep
    bias_row = (v1 * jnp.dot(c1, b2_ref[...], preferred_element_type=f32)
                + v2 * jnp.dot(c2, b2_ref[...], preferred_element_type=f32))

    cp_w1.wait()
    h = jnp.maximum(jnp.dot(x, w1_s[...], preferred_element_type=f32)
                    + b1_ref[...], 0.0)
    cp_w2.wait()
    routed = jnp.dot(h * scale, w2_s[...], preferred_element_type=f32)

    cp_sw1a.wait()
    sh0 = jnp.maximum(jnp.dot(x, sw1_s[0], preferred_element_type=f32)
                      + sb1_ref[0:1, :], 0.0)
    cp_sw2a.wait()
    acc = jnp.dot(sh0, sw2_s[0], preferred_element_type=f32) + sb2_ref[0:1, :]
    cp_sw1b.wait()
    sh1 = jnp.maximum(jnp.dot(x, sw1_s[1], preferred_element_type=f32)
                      + sb1_ref[1:2, :], 0.0)
    cp_sw2b.wait()
    acc = acc + jnp.dot(sh1, sw2_s[1], preferred_element_type=f32) + sb2_ref[1:2, :]

    o_ref[...] = acc + routed + bias_row


def kernel(x, gate_w, gate_b, W1, B1, W2, B2, SW1, SB1, SW2, SB2):
    b, s, d = x.shape
    x_flat = x.reshape(-1, d)
    w1cat = jnp.transpose(W1, (1, 0, 2)).reshape(d, _E * _HID)
    b1cat = B1.reshape(1, _E * _HID)
    w2cat = W2.reshape(_E * _HID, d)
    gb = gate_b.reshape(1, -1)
    f32 = jnp.float32

    vmem = pl.BlockSpec(memory_space=pltpu.MemorySpace.VMEM)
    hbm = pl.BlockSpec(memory_space=pltpu.MemorySpace.HBM)

    out = pl.pallas_call(
        _moe_body,
        out_shape=jax.ShapeDtypeStruct((x_flat.shape[0], d), f32),
        in_specs=[vmem, vmem, vmem, hbm, vmem, hbm, vmem,
                  hbm, vmem, hbm, vmem],
        out_specs=vmem,
        scratch_shapes=[
            pltpu.VMEM((d, _E * _HID), f32),
            pltpu.VMEM((_E * _HID, d), f32),
            pltpu.VMEM((2, d, d), f32),
            pltpu.VMEM((2, d, d), f32),
            pltpu.SemaphoreType.DMA((6,)),
        ],
    )(x_flat, gate_w, gb, w1cat, b1cat, w2cat, B2,
      SW1, SB1, SW2, SB2)
    return out.reshape(b, s, d)
